# baseline (speedup 1.0000x reference)
import functools

import jax
import jax.numpy as jnp
from jax import lax
from jax.experimental import pallas as pl
from jax.experimental.pallas import tpu as pltpu
from jax.experimental.pallas import tpu_sc as plsc

_C = 8
_G = 8
_B = _C * _G
_D = 500
_R = _G + _C - 1
_NW = 32
_APW = 2
_RPW = _APW * _R
_E = 32


@functools.cache
def _build_sc_gather():
    mesh = plsc.VectorSubcoreMesh(core_axis_name="c", subcore_axis_name="s")

    @functools.partial(
        pl.kernel,
        mesh=mesh,
        out_type=jax.ShapeDtypeStruct((_B * _R, 2 * _D), jnp.float32),
        scratch_types=[
            pltpu.VMEM((_APW * _E,), jnp.int32),
            pltpu.VMEM((_B, _D), jnp.float32),
            pltpu.VMEM((_RPW, 2 * _D), jnp.float32),
        ],
        compiler_params=pltpu.CompilerParams(
            use_tc_tiling_on_sc=False, needs_layout_passes=False),
    )
    def _sc_gather(table_hbm, gidx_hbm, out_hbm, idx_v, table_v, packed_v):
        wid = lax.axis_index("s") * 2 + lax.axis_index("c")
        pltpu.sync_copy(gidx_hbm.at[pl.ds(wid * _APW * _E, _APW * _E)], idx_v)
        pltpu.sync_copy(table_hbm, table_v)

        lanes = lax.broadcasted_iota(jnp.int32, (16,), 0)
        blocks = []
        for a in range(_APW):
            for r0 in (0, 14):
                e16 = plsc.load_gather(idx_v, [lanes + (a * _E + r0)])
                r = a * 2 * _R + r0 + lanes
                dst_row = r // 2
                dst_colb = (r % 2) * _D
                blocks.append((e16, dst_row, dst_colb))

        @plsc.parallel_loop(0, _D, unroll=8)
        def _assemble(c):
            col = jnp.full((16,), c, dtype=jnp.int32)
            for e16, dst_row, dst_colb in blocks:
                v = plsc.load_gather(table_v, [e16, col])
                plsc.store_scatter(packed_v, [dst_row, dst_colb + col], v)

        pltpu.sync_copy(packed_v, out_hbm.at[pl.ds(wid * _RPW, _RPW)])

    return _sc_gather


def kernel(inputs, targets):
    anchor_class = targets.astype(jnp.int32)
    pos_src = anchor_class[:, None] * _G + jnp.arange(_G, dtype=jnp.int32)[None, :]
    idx = jnp.arange(_C - 1, dtype=jnp.int32)
    neg_cls = idx[None, :] + (idx[None, :] >= anchor_class[:, None]).astype(jnp.int32)
    j = jax.random.randint(jax.random.key(1), (_B, _C - 1), 1, _G)
    neg_src = neg_cls * _G + j.astype(jnp.int32)
    partners = jnp.concatenate(
        [pos_src, neg_src, jnp.zeros((_B, 1), jnp.int32)], axis=1)
    anchors = jnp.broadcast_to(
        jnp.arange(_B, dtype=jnp.int32)[:, None], (_B, _E // 2))
    gidx = jnp.stack([anchors, partners], axis=-1).reshape(-1)

    expanded = _build_sc_gather()(inputs, gidx)

    labels = jnp.concatenate(
        [jnp.ones((_G,), jnp.int32), jnp.zeros((_C - 1,), jnp.int32)])
    new_targets = jnp.tile(labels, (_B,))
    return new_targets, expanded


# speedup vs baseline: 1.0501x; 1.0501x over previous
import functools

import jax
import jax.numpy as jnp
from jax import lax
from jax.experimental import pallas as pl
from jax.experimental.pallas import tpu as pltpu
from jax.experimental.pallas import tpu_sc as plsc

_C = 8
_G = 8
_B = _C * _G
_D = 500
_R = _G + _C - 1
_TOT = 2 * _B * _R
_ROWS = _B * _R
_SLAB = 32
_NW_USED = _ROWS // _SLAB
_EPW = 2 * _SLAB


@functools.cache
def _build_sc_gather():
    mesh = plsc.VectorSubcoreMesh(core_axis_name="c", subcore_axis_name="s")

    @functools.partial(
        pl.kernel,
        mesh=mesh,
        out_type=jax.ShapeDtypeStruct((_ROWS, 2 * _D), jnp.float32),
        scratch_types=[
            pltpu.VMEM((_TOT,), jnp.int32),
            pltpu.VMEM((_B, _D), jnp.float32),
            pltpu.VMEM((_SLAB, 2 * _D), jnp.float32),
        ],
        compiler_params=pltpu.CompilerParams(needs_layout_passes=False),
    )
    def _sc_gather(table_hbm, gidx_hbm, out_hbm, idx_v, table_v, packed_v):
        wid = lax.axis_index("s") * 2 + lax.axis_index("c")

        @pl.when(wid < _NW_USED)
        def _():
            pltpu.sync_copy(gidx_hbm, idx_v)
            pltpu.sync_copy(table_hbm, table_v)

            lanes = lax.broadcasted_iota(jnp.int32, (16,), 0)
            blocks = []
            for k in range(_EPW // 16):
                r = k * 16 + lanes
                e16 = plsc.load_gather(idx_v, [wid * _EPW + r])
                blocks.append((e16, r // 2, (r % 2) * _D))

            @plsc.parallel_loop(0, _D, unroll=4)
            def _assemble(c):
                col = jnp.full((16,), c, dtype=jnp.int32)
                for e16, dst_row, dst_colb in blocks:
                    v = plsc.load_gather(table_v, [e16, col])
                    plsc.store_scatter(packed_v, [dst_row, dst_colb + col], v)

            pltpu.sync_copy(packed_v, out_hbm.at[pl.ds(wid * _SLAB, _SLAB)])

    return _sc_gather


def kernel(inputs, targets):
    anchor_class = targets.astype(jnp.int32)
    pos_src = anchor_class[:, None] * _G + jnp.arange(_G, dtype=jnp.int32)[None, :]
    idx = jnp.arange(_C - 1, dtype=jnp.int32)
    neg_cls = idx[None, :] + (idx[None, :] >= anchor_class[:, None]).astype(jnp.int32)
    j = jax.random.randint(jax.random.key(1), (_B, _C - 1), 1, _G)
    neg_src = neg_cls * _G + j.astype(jnp.int32)
    partners = jnp.concatenate([pos_src, neg_src], axis=1)
    anchors = jnp.broadcast_to(
        jnp.arange(_B, dtype=jnp.int32)[:, None], (_B, _R))
    gidx = jnp.stack([anchors, partners], axis=-1).reshape(-1)

    expanded = _build_sc_gather()(inputs, gidx)

    labels = jnp.concatenate(
        [jnp.ones((_G,), jnp.int32), jnp.zeros((_C - 1,), jnp.int32)])
    new_targets = jnp.tile(labels, (_B,))
    return new_targets, expanded


# speedup vs baseline: 1.3266x; 1.2633x over previous
import functools

import jax
import jax.numpy as jnp
import numpy as np
from jax import lax
from jax.experimental import pallas as pl
from jax.experimental.pallas import tpu as pltpu
from jax.experimental.pallas import tpu_sc as plsc

_C = 8
_G = 8
_B = _C * _G
_D = 500
_R = _G + _C - 1
_ROWS = _B * _R
_SLAB = 32
_NW_USED = _ROWS // _SLAB
_EPW = 2 * _SLAB
_CHUNKS = tuple(range(0, _D - 16, 16)) + (_D - 16,)

_J = np.array([
    3, 2, 6, 6, 3, 7, 6, 6, 2, 6, 3, 6, 2, 3, 2, 5, 1, 3, 7, 3, 3, 2, 7, 6,
    3, 2, 7, 3, 4, 4, 5, 4, 6, 6, 4, 3, 1, 4, 3, 2, 5, 1, 1, 6, 4, 6, 4, 5,
    5, 7, 3, 3, 3, 4, 4, 2, 3, 3, 5, 5, 7, 2, 1, 4, 4, 4, 3, 1, 7, 2, 3, 7,
    4, 7, 4, 1, 1, 2, 4, 4, 7, 7, 6, 7, 5, 6, 1, 6, 1, 7, 7, 1, 6, 6, 5, 3,
    7, 4, 4, 6, 4, 6, 2, 6, 3, 4, 3, 3, 4, 6, 6, 2, 5, 7, 5, 4, 5, 1, 7, 2,
    4, 6, 1, 7, 1, 6, 5, 3, 1, 6, 3, 5, 3, 6, 6, 7, 5, 6, 5, 5, 7, 5, 7, 1,
    6, 2, 2, 3, 4, 4, 2, 1, 4, 4, 3, 1, 6, 7, 4, 5, 7, 6, 5, 5, 6, 4, 7, 5,
    5, 2, 4, 3, 5, 4, 6, 1, 6, 4, 3, 4, 7, 7, 1, 4, 7, 7, 2, 3, 6, 4, 2, 5,
    1, 4, 4, 3, 4, 4, 5, 4, 6, 1, 4, 5, 6, 3, 1, 7, 2, 1, 7, 1, 5, 1, 6, 4,
    3, 5, 2, 5, 1, 1, 4, 6, 2, 3, 4, 2, 1, 7, 4, 1, 5, 7, 2, 2, 2, 7, 6, 5,
    2, 5, 3, 4, 6, 3, 7, 3, 6, 3, 4, 3, 6, 5, 7, 3, 5, 5, 7, 2, 6, 1, 4, 6,
    5, 6, 6, 2, 2, 1, 5, 2, 7, 6, 3, 4, 3, 2, 3, 7, 3, 2, 7, 3, 6, 2, 4, 3,
    6, 2, 2, 5, 3, 1, 4, 5, 3, 5, 3, 4, 7, 2, 6, 5, 5, 3, 5, 2, 4, 3, 5, 5,
    1, 4, 2, 7, 6, 3, 6, 5, 7, 6, 4, 1, 3, 4, 7, 7, 7, 4, 4, 7, 4, 4, 6, 3,
    3, 3, 3, 3, 1, 3, 3, 6, 1, 3, 2, 3, 6, 6, 5, 3, 7, 2, 2, 5, 2, 5, 2, 5,
    6, 3, 3, 5, 5, 5, 2, 6, 4, 3, 7, 6, 1, 1, 6, 6, 5, 1, 2, 7, 5, 5, 1, 7,
    6, 6, 5, 4, 3, 3, 3, 4, 6, 2, 4, 4, 3, 4, 7, 4, 4, 6, 2, 4, 3, 2, 6, 6,
    1, 5, 1, 4, 1, 3, 3, 3, 3, 7, 4, 7, 5, 2, 3, 6, 6, 6, 4, 7, 4, 5, 3, 4,
    1, 5, 4, 1, 4, 5, 6, 2, 2, 5, 1, 2, 1, 3, 4, 2,
], dtype=np.int32)


@functools.cache
def _build_sc_gather():
    mesh = plsc.VectorSubcoreMesh(core_axis_name="c", subcore_axis_name="s")

    @functools.partial(
        pl.kernel,
        mesh=mesh,
        out_type=jax.ShapeDtypeStruct((_ROWS, 2 * _D), jnp.float32),
        scratch_types=[
            pltpu.VMEM((_B,), jnp.int32),
            pltpu.VMEM((_B * (_C - 1),), jnp.int32),
            pltpu.VMEM((_EPW,), jnp.int32),
            pltpu.VMEM((len(_CHUNKS) * 16,), jnp.int32),
            pltpu.VMEM((_B, _D), jnp.float32),
            pltpu.VMEM((_SLAB, 2 * _D), jnp.float32),
            pltpu.SemaphoreType.DMA,
            pltpu.SemaphoreType.DMA,
        ],
        compiler_params=pltpu.CompilerParams(needs_layout_passes=False),
    )
    def _sc_gather(table_hbm, targets_hbm, j_hbm, out_hbm,
                   tgt_v, j_v, srow_v, col_v, table_v, packed_v, sem, sem2):
        wid = lax.axis_index("s") * 2 + lax.axis_index("c")

        @pl.when(wid < _NW_USED)
        def _():
            table_cp = pltpu.async_copy(table_hbm, table_v, sem)
            pltpu.sync_copy(targets_hbm, tgt_v)
            pltpu.sync_copy(j_hbm, j_v)

            lanes = lax.broadcasted_iota(jnp.int32, (16,), 0)
            for k in range(_EPW // 16):
                r = wid * _EPW + k * 16 + lanes
                b = r // 30
                rr = r - b * 30
                pair = rr // 2
                odd = rr - pair * 2
                tc = plsc.load_gather(tgt_v, [b])
                i = pair - _G
                jj = plsc.load_gather(j_v, [b * (_C - 1) + jnp.maximum(i, 0)])
                ncls = i + (i >= tc).astype(jnp.int32)
                part = jnp.where(i >= 0, ncls * _G + jj, tc * _G + pair)
                srow_v[pl.ds(k * 16, 16)] = jnp.where(odd == 1, part, b)
            for ci, cc in enumerate(_CHUNKS):
                col_v[pl.ds(ci * 16, 16)] = cc + lanes

            table_cp.wait()

            def _row(rl, half):
                rp = half * _SLAB + rl
                srow = plsc.load_gather(srow_v, [jnp.full((16,), rp, jnp.int32)])
                p = rp // 2
                colb = (rp - p * 2) * _D
                prow = jnp.full((16,), p, jnp.int32)
                cb = jnp.full((16,), colb, jnp.int32)
                for ci in range(len(_CHUNKS)):
                    colv = col_v[pl.ds(ci * 16, 16)]
                    v = plsc.load_gather(table_v, [srow, colv])
                    plsc.store_scatter(packed_v, [prow, cb + colv], v)

            half_rows = _SLAB // 2
            for half in range(2):
                plsc.parallel_loop(0, _SLAB, unroll=1)(
                    functools.partial(_row, half=half))
                cp = pltpu.async_copy(
                    packed_v.at[pl.ds(half * half_rows, half_rows)],
                    out_hbm.at[pl.ds(wid * _SLAB + half * half_rows, half_rows)],
                    sem2)
                if half == 1:
                    cp.wait()
                else:
                    _pending = cp
            _pending.wait()

    return _sc_gather


def kernel(inputs, targets):
    jconst = jnp.asarray(_J)
    expanded = _build_sc_gather()(inputs, targets.astype(jnp.int32), jconst)

    labels = jnp.concatenate(
        [jnp.ones((_G,), jnp.int32), jnp.zeros((_C - 1,), jnp.int32)])
    new_targets = jnp.tile(labels, (_B,))
    return new_targets, expanded


# speedup vs baseline: 1.5114x; 1.1393x over previous
import functools

import jax
import jax.numpy as jnp
import numpy as np
from jax import lax
from jax.experimental import pallas as pl
from jax.experimental.pallas import tpu as pltpu
from jax.experimental.pallas import tpu_sc as plsc

_C = 8
_G = 8
_B = _C * _G
_D = 500
_R = _G + _C - 1
_ROWS = _B * _R
_SLAB = 32
_NW_USED = _ROWS // _SLAB
_EPW = 2 * _SLAB
_CHUNKS = tuple(range(0, _D - 16, 16)) + (_D - 16,)

_J = np.array([
    3, 2, 6, 6, 3, 7, 6, 6, 2, 6, 3, 6, 2, 3, 2, 5, 1, 3, 7, 3, 3, 2, 7, 6,
    3, 2, 7, 3, 4, 4, 5, 4, 6, 6, 4, 3, 1, 4, 3, 2, 5, 1, 1, 6, 4, 6, 4, 5,
    5, 7, 3, 3, 3, 4, 4, 2, 3, 3, 5, 5, 7, 2, 1, 4, 4, 4, 3, 1, 7, 2, 3, 7,
    4, 7, 4, 1, 1, 2, 4, 4, 7, 7, 6, 7, 5, 6, 1, 6, 1, 7, 7, 1, 6, 6, 5, 3,
    7, 4, 4, 6, 4, 6, 2, 6, 3, 4, 3, 3, 4, 6, 6, 2, 5, 7, 5, 4, 5, 1, 7, 2,
    4, 6, 1, 7, 1, 6, 5, 3, 1, 6, 3, 5, 3, 6, 6, 7, 5, 6, 5, 5, 7, 5, 7, 1,
    6, 2, 2, 3, 4, 4, 2, 1, 4, 4, 3, 1, 6, 7, 4, 5, 7, 6, 5, 5, 6, 4, 7, 5,
    5, 2, 4, 3, 5, 4, 6, 1, 6, 4, 3, 4, 7, 7, 1, 4, 7, 7, 2, 3, 6, 4, 2, 5,
    1, 4, 4, 3, 4, 4, 5, 4, 6, 1, 4, 5, 6, 3, 1, 7, 2, 1, 7, 1, 5, 1, 6, 4,
    3, 5, 2, 5, 1, 1, 4, 6, 2, 3, 4, 2, 1, 7, 4, 1, 5, 7, 2, 2, 2, 7, 6, 5,
    2, 5, 3, 4, 6, 3, 7, 3, 6, 3, 4, 3, 6, 5, 7, 3, 5, 5, 7, 2, 6, 1, 4, 6,
    5, 6, 6, 2, 2, 1, 5, 2, 7, 6, 3, 4, 3, 2, 3, 7, 3, 2, 7, 3, 6, 2, 4, 3,
    6, 2, 2, 5, 3, 1, 4, 5, 3, 5, 3, 4, 7, 2, 6, 5, 5, 3, 5, 2, 4, 3, 5, 5,
    1, 4, 2, 7, 6, 3, 6, 5, 7, 6, 4, 1, 3, 4, 7, 7, 7, 4, 4, 7, 4, 4, 6, 3,
    3, 3, 3, 3, 1, 3, 3, 6, 1, 3, 2, 3, 6, 6, 5, 3, 7, 2, 2, 5, 2, 5, 2, 5,
    6, 3, 3, 5, 5, 5, 2, 6, 4, 3, 7, 6, 1, 1, 6, 6, 5, 1, 2, 7, 5, 5, 1, 7,
    6, 6, 5, 4, 3, 3, 3, 4, 6, 2, 4, 4, 3, 4, 7, 4, 4, 6, 2, 4, 3, 2, 6, 6,
    1, 5, 1, 4, 1, 3, 3, 3, 3, 7, 4, 7, 5, 2, 3, 6, 6, 6, 4, 7, 4, 5, 3, 4,
    1, 5, 4, 1, 4, 5, 6, 2, 2, 5, 1, 2, 1, 3, 4, 2,
], dtype=np.int32)

_NEW_TARGETS = np.tile(
    np.array([1] * _G + [0] * (_C - 1), dtype=np.int32), _B)


@functools.cache
def _build_sc_gather():
    mesh = plsc.VectorSubcoreMesh(core_axis_name="c", subcore_axis_name="s")

    @functools.partial(
        pl.kernel,
        mesh=mesh,
        out_type=jax.ShapeDtypeStruct((_ROWS, 2 * _D), jnp.float32),
        scratch_types=[
            pltpu.VMEM((_B,), jnp.int32),
            pltpu.VMEM((_B * (_C - 1),), jnp.int32),
            pltpu.VMEM((_EPW,), jnp.int32),
            pltpu.VMEM((len(_CHUNKS) * 16,), jnp.int32),
            pltpu.VMEM((_B, _D), jnp.float32),
            pltpu.VMEM((_SLAB, 2 * _D), jnp.float32),
            pltpu.SemaphoreType.DMA,
            pltpu.SemaphoreType.DMA,
        ],
        compiler_params=pltpu.CompilerParams(needs_layout_passes=False),
    )
    def _sc_gather(table_hbm, targets_hbm, j_hbm, out_hbm,
                   tgt_v, j_v, srow_v, col_v, table_v, packed_v, sem, sem2):
        wid = lax.axis_index("s") * 2 + lax.axis_index("c")

        @pl.when(wid < _NW_USED)
        def _():
            table_cp = pltpu.async_copy(table_hbm, table_v, sem)
            pltpu.sync_copy(targets_hbm, tgt_v)
            pltpu.sync_copy(j_hbm, j_v)

            lanes = lax.broadcasted_iota(jnp.int32, (16,), 0)
            for k in range(_EPW // 16):
                r = wid * _EPW + k * 16 + lanes
                b = r // 30
                rr = r - b * 30
                pair = rr // 2
                odd = rr - pair * 2
                tc = plsc.load_gather(tgt_v, [b])
                i = pair - _G
                jj = plsc.load_gather(j_v, [b * (_C - 1) + jnp.maximum(i, 0)])
                ncls = i + (i >= tc).astype(jnp.int32)
                part = jnp.where(i >= 0, ncls * _G + jj, tc * _G + pair)
                srow_v[pl.ds(k * 16, 16)] = jnp.where(odd == 1, part, b)
            for ci, cc in enumerate(_CHUNKS):
                col_v[pl.ds(ci * 16, 16)] = cc + lanes

            table_cp.wait()

            def _row(rl, half):
                rp = half * _SLAB + rl
                srow = plsc.load_gather(srow_v, [jnp.full((16,), rp, jnp.int32)])
                p = rp // 2
                colb = (rp - p * 2) * _D
                prow = jnp.full((16,), p, jnp.int32)
                cb = jnp.full((16,), colb, jnp.int32)

                @plsc.parallel_loop(0, len(_CHUNKS), unroll=4)
                def _chunk(ci):
                    colv = col_v[pl.ds(ci * 16, 16)]
                    v = plsc.load_gather(table_v, [srow, colv])
                    plsc.store_scatter(packed_v, [prow, cb + colv], v)

            half_rows = _SLAB // 2
            for half in range(2):
                plsc.parallel_loop(0, _SLAB, unroll=1)(
                    functools.partial(_row, half=half))
                cp = pltpu.async_copy(
                    packed_v.at[pl.ds(half * half_rows, half_rows)],
                    out_hbm.at[pl.ds(wid * _SLAB + half * half_rows, half_rows)],
                    sem2)
                if half == 1:
                    cp.wait()
                else:
                    _pending = cp
            _pending.wait()

    return _sc_gather


def kernel(inputs, targets):
    jconst = jnp.asarray(_J)
    expanded = _build_sc_gather()(inputs, targets.astype(jnp.int32), jconst)

    new_targets = jnp.asarray(_NEW_TARGETS)
    return new_targets, expanded


# speedup vs baseline: 1.5155x; 1.0027x over previous
import functools

import jax
import jax.numpy as jnp
import numpy as np
from jax import lax
from jax.experimental import pallas as pl
from jax.experimental.pallas import tpu as pltpu
from jax.experimental.pallas import tpu_sc as plsc

_C = 8
_G = 8
_B = _C * _G
_D = 500
_R = _G + _C - 1
_ROWS = _B * _R
_SLAB = 32
_NW_USED = _ROWS // _SLAB
_EPW = 2 * _SLAB
_CHUNKS = tuple(range(0, _D - 16, 16)) + (_D - 16,)

_J = np.array([
    3, 2, 6, 6, 3, 7, 6, 6, 2, 6, 3, 6, 2, 3, 2, 5, 1, 3, 7, 3, 3, 2, 7, 6,
    3, 2, 7, 3, 4, 4, 5, 4, 6, 6, 4, 3, 1, 4, 3, 2, 5, 1, 1, 6, 4, 6, 4, 5,
    5, 7, 3, 3, 3, 4, 4, 2, 3, 3, 5, 5, 7, 2, 1, 4, 4, 4, 3, 1, 7, 2, 3, 7,
    4, 7, 4, 1, 1, 2, 4, 4, 7, 7, 6, 7, 5, 6, 1, 6, 1, 7, 7, 1, 6, 6, 5, 3,
    7, 4, 4, 6, 4, 6, 2, 6, 3, 4, 3, 3, 4, 6, 6, 2, 5, 7, 5, 4, 5, 1, 7, 2,
    4, 6, 1, 7, 1, 6, 5, 3, 1, 6, 3, 5, 3, 6, 6, 7, 5, 6, 5, 5, 7, 5, 7, 1,
    6, 2, 2, 3, 4, 4, 2, 1, 4, 4, 3, 1, 6, 7, 4, 5, 7, 6, 5, 5, 6, 4, 7, 5,
    5, 2, 4, 3, 5, 4, 6, 1, 6, 4, 3, 4, 7, 7, 1, 4, 7, 7, 2, 3, 6, 4, 2, 5,
    1, 4, 4, 3, 4, 4, 5, 4, 6, 1, 4, 5, 6, 3, 1, 7, 2, 1, 7, 1, 5, 1, 6, 4,
    3, 5, 2, 5, 1, 1, 4, 6, 2, 3, 4, 2, 1, 7, 4, 1, 5, 7, 2, 2, 2, 7, 6, 5,
    2, 5, 3, 4, 6, 3, 7, 3, 6, 3, 4, 3, 6, 5, 7, 3, 5, 5, 7, 2, 6, 1, 4, 6,
    5, 6, 6, 2, 2, 1, 5, 2, 7, 6, 3, 4, 3, 2, 3, 7, 3, 2, 7, 3, 6, 2, 4, 3,
    6, 2, 2, 5, 3, 1, 4, 5, 3, 5, 3, 4, 7, 2, 6, 5, 5, 3, 5, 2, 4, 3, 5, 5,
    1, 4, 2, 7, 6, 3, 6, 5, 7, 6, 4, 1, 3, 4, 7, 7, 7, 4, 4, 7, 4, 4, 6, 3,
    3, 3, 3, 3, 1, 3, 3, 6, 1, 3, 2, 3, 6, 6, 5, 3, 7, 2, 2, 5, 2, 5, 2, 5,
    6, 3, 3, 5, 5, 5, 2, 6, 4, 3, 7, 6, 1, 1, 6, 6, 5, 1, 2, 7, 5, 5, 1, 7,
    6, 6, 5, 4, 3, 3, 3, 4, 6, 2, 4, 4, 3, 4, 7, 4, 4, 6, 2, 4, 3, 2, 6, 6,
    1, 5, 1, 4, 1, 3, 3, 3, 3, 7, 4, 7, 5, 2, 3, 6, 6, 6, 4, 7, 4, 5, 3, 4,
    1, 5, 4, 1, 4, 5, 6, 2, 2, 5, 1, 2, 1, 3, 4, 2,
], dtype=np.int32)

_NEW_TARGETS = np.tile(
    np.array([1] * _G + [0] * (_C - 1), dtype=np.int32), _B)


@functools.cache
def _build_sc_gather():
    mesh = plsc.VectorSubcoreMesh(core_axis_name="c", subcore_axis_name="s")

    @functools.partial(
        pl.kernel,
        mesh=mesh,
        out_type=jax.ShapeDtypeStruct((_ROWS, 2 * _D), jnp.float32),
        scratch_types=[
            pltpu.VMEM((_B,), jnp.int32),
            pltpu.VMEM((_B * (_C - 1),), jnp.int32),
            pltpu.VMEM((_EPW,), jnp.int32),
            pltpu.VMEM((len(_CHUNKS) * 16,), jnp.int32),
            pltpu.VMEM((_B, _D), jnp.float32),
            pltpu.VMEM((_SLAB, 2 * _D), jnp.float32),
            pltpu.SemaphoreType.DMA,
            pltpu.SemaphoreType.DMA,
        ],
        compiler_params=pltpu.CompilerParams(needs_layout_passes=False),
    )
    def _sc_gather(table_hbm, targets_hbm, j_hbm, out_hbm,
                   tgt_v, j_v, srow_v, col_v, table_v, packed_v, sem, sem2):
        wid = lax.axis_index("s") * 2 + lax.axis_index("c")

        @pl.when(wid < _NW_USED)
        def _():
            table_cp = pltpu.async_copy(table_hbm, table_v, sem)
            pltpu.sync_copy(targets_hbm, tgt_v)
            pltpu.sync_copy(j_hbm, j_v)

            lanes = lax.broadcasted_iota(jnp.int32, (16,), 0)
            for k in range(_EPW // 16):
                r = wid * _EPW + k * 16 + lanes
                b = r // 30
                rr = r - b * 30
                pair = rr // 2
                odd = rr - pair * 2
                tc = plsc.load_gather(tgt_v, [b])
                i = pair - _G
                jj = plsc.load_gather(j_v, [b * (_C - 1) + jnp.maximum(i, 0)])
                ncls = i + (i >= tc).astype(jnp.int32)
                part = jnp.where(i >= 0, ncls * _G + jj, tc * _G + pair)
                srow_v[pl.ds(k * 16, 16)] = jnp.where(odd == 1, part, b)
            for ci, cc in enumerate(_CHUNKS):
                col_v[pl.ds(ci * 16, 16)] = cc + lanes

            table_cp.wait()

            def _row(rl, half):
                rp = half * _SLAB + rl
                srow = plsc.load_gather(srow_v, [jnp.full((16,), rp, jnp.int32)])
                p = rp // 2
                colb = (rp - p * 2) * _D
                prow = jnp.full((16,), p, jnp.int32)
                cb = jnp.full((16,), colb, jnp.int32)

                @plsc.parallel_loop(0, len(_CHUNKS), unroll=8)
                def _chunk(ci):
                    colv = col_v[pl.ds(ci * 16, 16)]
                    v = plsc.load_gather(table_v, [srow, colv])
                    plsc.store_scatter(packed_v, [prow, cb + colv], v)

            half_rows = _SLAB // 2
            for half in range(2):
                plsc.parallel_loop(0, _SLAB, unroll=1)(
                    functools.partial(_row, half=half))
                cp = pltpu.async_copy(
                    packed_v.at[pl.ds(half * half_rows, half_rows)],
                    out_hbm.at[pl.ds(wid * _SLAB + half * half_rows, half_rows)],
                    sem2)
                if half == 1:
                    cp.wait()
                else:
                    _pending = cp
            _pending.wait()

    return _sc_gather


def kernel(inputs, targets):
    jconst = jnp.asarray(_J)
    expanded = _build_sc_gather()(inputs, targets.astype(jnp.int32), jconst)

    new_targets = jnp.asarray(_NEW_TARGETS)
    return new_targets, expanded


# speedup vs baseline: 1.5464x; 1.0204x over previous
import functools

import jax
import jax.numpy as jnp
import numpy as np
from jax import lax
from jax.experimental import pallas as pl
from jax.experimental.pallas import tpu as pltpu
from jax.experimental.pallas import tpu_sc as plsc

_C = 8
_G = 8
_B = _C * _G
_D = 500
_R = _G + _C - 1
_ROWS = _B * _R
_SLAB = 32
_NW_USED = _ROWS // _SLAB
_EPW = 2 * _SLAB
_CHUNKS = tuple(range(0, _D - 16, 16)) + (_D - 16,)

_J = np.array([
    3, 2, 6, 6, 3, 7, 6, 6, 2, 6, 3, 6, 2, 3, 2, 5, 1, 3, 7, 3, 3, 2, 7, 6,
    3, 2, 7, 3, 4, 4, 5, 4, 6, 6, 4, 3, 1, 4, 3, 2, 5, 1, 1, 6, 4, 6, 4, 5,
    5, 7, 3, 3, 3, 4, 4, 2, 3, 3, 5, 5, 7, 2, 1, 4, 4, 4, 3, 1, 7, 2, 3, 7,
    4, 7, 4, 1, 1, 2, 4, 4, 7, 7, 6, 7, 5, 6, 1, 6, 1, 7, 7, 1, 6, 6, 5, 3,
    7, 4, 4, 6, 4, 6, 2, 6, 3, 4, 3, 3, 4, 6, 6, 2, 5, 7, 5, 4, 5, 1, 7, 2,
    4, 6, 1, 7, 1, 6, 5, 3, 1, 6, 3, 5, 3, 6, 6, 7, 5, 6, 5, 5, 7, 5, 7, 1,
    6, 2, 2, 3, 4, 4, 2, 1, 4, 4, 3, 1, 6, 7, 4, 5, 7, 6, 5, 5, 6, 4, 7, 5,
    5, 2, 4, 3, 5, 4, 6, 1, 6, 4, 3, 4, 7, 7, 1, 4, 7, 7, 2, 3, 6, 4, 2, 5,
    1, 4, 4, 3, 4, 4, 5, 4, 6, 1, 4, 5, 6, 3, 1, 7, 2, 1, 7, 1, 5, 1, 6, 4,
    3, 5, 2, 5, 1, 1, 4, 6, 2, 3, 4, 2, 1, 7, 4, 1, 5, 7, 2, 2, 2, 7, 6, 5,
    2, 5, 3, 4, 6, 3, 7, 3, 6, 3, 4, 3, 6, 5, 7, 3, 5, 5, 7, 2, 6, 1, 4, 6,
    5, 6, 6, 2, 2, 1, 5, 2, 7, 6, 3, 4, 3, 2, 3, 7, 3, 2, 7, 3, 6, 2, 4, 3,
    6, 2, 2, 5, 3, 1, 4, 5, 3, 5, 3, 4, 7, 2, 6, 5, 5, 3, 5, 2, 4, 3, 5, 5,
    1, 4, 2, 7, 6, 3, 6, 5, 7, 6, 4, 1, 3, 4, 7, 7, 7, 4, 4, 7, 4, 4, 6, 3,
    3, 3, 3, 3, 1, 3, 3, 6, 1, 3, 2, 3, 6, 6, 5, 3, 7, 2, 2, 5, 2, 5, 2, 5,
    6, 3, 3, 5, 5, 5, 2, 6, 4, 3, 7, 6, 1, 1, 6, 6, 5, 1, 2, 7, 5, 5, 1, 7,
    6, 6, 5, 4, 3, 3, 3, 4, 6, 2, 4, 4, 3, 4, 7, 4, 4, 6, 2, 4, 3, 2, 6, 6,
    1, 5, 1, 4, 1, 3, 3, 3, 3, 7, 4, 7, 5, 2, 3, 6, 6, 6, 4, 7, 4, 5, 3, 4,
    1, 5, 4, 1, 4, 5, 6, 2, 2, 5, 1, 2, 1, 3, 4, 2,
], dtype=np.int32)

_NEW_TARGETS = np.tile(
    np.array([1] * _G + [0] * (_C - 1), dtype=np.int32), _B)


@functools.cache
def _build_sc_gather():
    mesh = plsc.VectorSubcoreMesh(core_axis_name="c", subcore_axis_name="s")

    @functools.partial(
        pl.kernel,
        mesh=mesh,
        out_type=jax.ShapeDtypeStruct((_ROWS, 2 * _D), jnp.float32),
        scratch_types=[
            pltpu.VMEM((_B,), jnp.int32),
            pltpu.VMEM((_B * (_C - 1),), jnp.int32),
            pltpu.VMEM((_EPW,), jnp.int32),
            pltpu.VMEM((len(_CHUNKS) * 16,), jnp.int32),
            pltpu.VMEM((_B, _D), jnp.float32),
            pltpu.VMEM((_SLAB, 2 * _D), jnp.float32),
            pltpu.SemaphoreType.DMA,
            pltpu.SemaphoreType.DMA,
        ],
        compiler_params=pltpu.CompilerParams(needs_layout_passes=False),
    )
    def _sc_gather(table_hbm, targets_hbm, j_hbm, out_hbm,
                   tgt_v, j_v, srow_v, col_v, table_v, packed_v, sem, sem2):
        wid = lax.axis_index("s") * 2 + lax.axis_index("c")

        @pl.when(wid < _NW_USED)
        def _():
            table_cp = pltpu.async_copy(table_hbm, table_v, sem)
            pltpu.sync_copy(targets_hbm, tgt_v)
            pltpu.sync_copy(j_hbm, j_v)

            lanes = lax.broadcasted_iota(jnp.int32, (16,), 0)
            for k in range(_EPW // 16):
                r = wid * _EPW + k * 16 + lanes
                b = r // 30
                rr = r - b * 30
                pair = rr // 2
                odd = rr - pair * 2
                tc = plsc.load_gather(tgt_v, [b])
                i = pair - _G
                jj = plsc.load_gather(j_v, [b * (_C - 1) + jnp.maximum(i, 0)])
                ncls = i + (i >= tc).astype(jnp.int32)
                part = jnp.where(i >= 0, ncls * _G + jj, tc * _G + pair)
                srow_v[pl.ds(k * 16, 16)] = jnp.where(odd == 1, part, b)
            for ci, cc in enumerate(_CHUNKS):
                col_v[pl.ds(ci * 16, 16)] = cc + lanes

            table_cp.wait()

            def _row(rl, half):
                rp = half * _SLAB + rl
                srow = plsc.load_gather(srow_v, [jnp.full((16,), rp, jnp.int32)])
                p = rp // 2
                colb = (rp - p * 2) * _D
                prow = jnp.full((16,), p, jnp.int32)
                cb = jnp.full((16,), colb, jnp.int32)

                @plsc.parallel_loop(0, len(_CHUNKS), unroll=8)
                def _chunk(ci):
                    colv = col_v[pl.ds(ci * 16, 16)]
                    v = plsc.load_gather(table_v, [srow, colv])
                    plsc.store_scatter(packed_v, [prow, cb + colv], v)

            half_rows = _SLAB // 2
            for half in range(2):
                plsc.parallel_loop(0, _SLAB, unroll=2)(
                    functools.partial(_row, half=half))
                cp = pltpu.async_copy(
                    packed_v.at[pl.ds(half * half_rows, half_rows)],
                    out_hbm.at[pl.ds(wid * _SLAB + half * half_rows, half_rows)],
                    sem2)
                if half == 1:
                    cp.wait()
                else:
                    _pending = cp
            _pending.wait()

    return _sc_gather


def kernel(inputs, targets):
    jconst = jnp.asarray(_J)
    expanded = _build_sc_gather()(inputs, targets.astype(jnp.int32), jconst)

    new_targets = jnp.asarray(_NEW_TARGETS)
    return new_targets, expanded
